# baseline (speedup 1.0000x reference)
import functools

import jax
import jax.numpy as jnp
from jax import lax
from jax.experimental import pallas as pl
from jax.experimental.pallas import tpu as pltpu
from jax.experimental.pallas import tpu_sc as plsc

BATCH = 16384
NUM_NUMERIC = 13
NUM_CATEGORICAL = 26
STATE = 64
CARD = 100000

NC = 2
NS = 16
NW = NC * NS
B_PER_W = BATCH // NW
G_CHUNK = 128
N_SUB = B_PER_W // G_CHUNK
NUM_COLS = NUM_NUMERIC * STATE
N_CHUNK = 32
NUM_NCHUNKS = B_PER_W // N_CHUNK


def _body(x_num, xcat_r, w_hbm, bias_hbm, emb_hbm, out_hbm,
          idx_v, gbuf, xc, w_v, b_v, num_buf, sem_g, sem_x):
    wid = lax.axis_index("s") * NC + lax.axis_index("c")
    base = wid * B_PER_W

    pltpu.sync_copy(xcat_r.at[:, pl.ds(wid * N_SUB, N_SUB), :], idx_v)

    def _off_body(n, _):
        t = n // (N_SUB * (G_CHUNK // 16))
        r = (n // (G_CHUNK // 16)) % N_SUB
        k = n % (G_CHUNK // 16)
        sl = idx_v[t, r, pl.ds(k * 16, 16)]
        idx_v[t, r, pl.ds(k * 16, 16)] = sl + t * CARD
        return 0
    lax.fori_loop(0, NUM_CATEGORICAL * N_SUB * (G_CHUNK // 16), _off_body, 0)

    pltpu.sync_copy(w_hbm, w_v)
    pltpu.sync_copy(bias_hbm, b_v)

    def _tab_body(t, _):
        copies = []
        for s in range(N_SUB):
            c = pltpu.make_async_copy(
                emb_hbm.at[idx_v.at[t, s]],
                gbuf.at[pl.ds(s * G_CHUNK, G_CHUNK)],
                sem_g,
            )
            c.start()
            copies.append(c)
        for c in copies:
            c.wait()
        pltpu.sync_copy(
            gbuf,
            out_hbm.at[pl.ds(base, B_PER_W),
                       pl.ds(NUM_COLS + t * STATE, STATE)],
        )
        return 0
    lax.fori_loop(0, NUM_CATEGORICAL, _tab_body, 0)

    def _nchunk_body(ch, _):
        row0 = base + ch * N_CHUNK
        pltpu.sync_copy(x_num.at[pl.ds(row0, N_CHUNK), :], xc)
        for i in range(NUM_NUMERIC):
            wv = [w_v[i, pl.ds(c * 16, 16)] for c in range(STATE // 16)]
            bv = [b_v[i, pl.ds(c * 16, 16)] for c in range(STATE // 16)]

            def _row_body(b, _, i=i, wv=wv, bv=bv):
                splat = plsc.load_gather(
                    xc, [jnp.full((16,), b, jnp.int32),
                         jnp.full((16,), i, jnp.int32)])
                for c in range(STATE // 16):
                    num_buf[b, pl.ds(i * STATE + c * 16, 16)] = (
                        splat * wv[c] + bv[c])
                return 0
            lax.fori_loop(0, N_CHUNK, _row_body, 0)
        pltpu.sync_copy(
            num_buf, out_hbm.at[pl.ds(row0, N_CHUNK), pl.ds(0, NUM_COLS)])
        return 0
    lax.fori_loop(0, NUM_NCHUNKS, _nchunk_body, 0)


@jax.jit
def _run(x_numeric, xcat_r, W_num, b_num, emb_flat):
    mesh = plsc.VectorSubcoreMesh(core_axis_name="c", subcore_axis_name="s")
    return pl.kernel(
        _body,
        mesh=mesh,
        compiler_params=pltpu.CompilerParams(use_tc_tiling_on_sc=False,
                                             needs_layout_passes=False),
        out_type=jax.ShapeDtypeStruct((BATCH, NUM_CATEGORICAL * STATE + NUM_COLS),
                                      jnp.float32),
        scratch_types=[
            pltpu.VMEM((NUM_CATEGORICAL, N_SUB, G_CHUNK), jnp.int32),
            pltpu.VMEM((B_PER_W, STATE), jnp.float32),
            pltpu.VMEM((N_CHUNK, NUM_NUMERIC), jnp.float32),
            pltpu.VMEM((NUM_NUMERIC, STATE), jnp.float32),
            pltpu.VMEM((NUM_NUMERIC, STATE), jnp.float32),
            pltpu.VMEM((N_CHUNK, NUM_COLS), jnp.float32),
            pltpu.SemaphoreType.DMA,
            pltpu.SemaphoreType.DMA,
        ],
    )(x_numeric, xcat_r, W_num, b_num, emb_flat)


def kernel(x_numeric, x_categorical, W_num, b_num, emb_tables):
    xcat_r = (x_categorical.astype(jnp.int32).T
              .reshape(NUM_CATEGORICAL, BATCH // G_CHUNK, G_CHUNK))
    emb_flat = emb_tables.reshape(NUM_CATEGORICAL * CARD, STATE)
    return _run(x_numeric, xcat_r, W_num, b_num, emb_flat)
